# Initial kernel scaffold; baseline (speedup 1.0000x reference)
import jax
import jax.numpy as jnp
from jax.experimental import pallas as pl


def kernel(x, edge_index, W1, b1, W2, b2, Wa1, ba1, Wa2, ba2, Wv1, bv1, Wv2, bv2):
    raise NotImplementedError("write your pallas kernel here")


# speedup vs baseline: 1.0665x; 1.0665x over previous
import functools

import jax
import jax.numpy as jnp
from jax.experimental import pallas as pl
from jax.experimental.pallas import tpu as pltpu

N = 10000
E = 320000
HID = 16
FINAL = HID * N


def _gcn_xla(x, edge_index, W, b):
    n = x.shape[0]
    src = edge_index[0]
    dst = edge_index[1]
    loop = jnp.arange(n, dtype=src.dtype)
    src = jnp.concatenate([src, loop])
    dst = jnp.concatenate([dst, loop])
    deg = jnp.zeros((n,), jnp.float32).at[dst].add(1.0)
    dinv = jax.lax.rsqrt(deg)
    norm = dinv[src] * dinv[dst]
    h = x @ W
    msg = h[src] * norm[:, None]
    out = jnp.zeros((n, W.shape[1]), jnp.float32).at[dst].add(msg)
    return out + b


_KB = 3200


def _head1_body(f_ref, wa_ref, wv_ref, oa_ref, ov_ref):
    @pl.when(pl.program_id(0) == 0)
    def _init():
        oa_ref[...] = jnp.zeros_like(oa_ref)
        ov_ref[...] = jnp.zeros_like(ov_ref)

    f = f_ref[...]
    oa_ref[...] += jnp.dot(f, wa_ref[...], preferred_element_type=jnp.float32)
    ov_ref[...] += jnp.dot(f, wv_ref[...], preferred_element_type=jnp.float32)


def _head1(flat, Wa1, Wv1):
    grid = (FINAL // _KB,)
    return pl.pallas_call(
        _head1_body,
        grid=grid,
        in_specs=[
            pl.BlockSpec((1, _KB), lambda i: (0, i)),
            pl.BlockSpec((_KB, 64), lambda i: (i, 0)),
            pl.BlockSpec((_KB, 64), lambda i: (i, 0)),
        ],
        out_specs=[
            pl.BlockSpec((1, 64), lambda i: (0, 0)),
            pl.BlockSpec((1, 64), lambda i: (0, 0)),
        ],
        out_shape=[
            jax.ShapeDtypeStruct((1, 64), jnp.float32),
            jax.ShapeDtypeStruct((1, 64), jnp.float32),
        ],
    )(flat, Wa1, Wv1)


def _head2_body(apre_ref, ba1_ref, wa2_ref, ba2_ref, vpre_ref, bv1_ref,
                wv2_ref, bv2_ref, act_ref, val_ref):
    a = jnp.maximum(apre_ref[...] + ba1_ref[...], 0.0)
    logits = jnp.sum(a.reshape(64, 1) * wa2_ref[...], axis=0,
                     keepdims=True) + ba2_ref[...]
    m = jnp.max(logits, axis=-1, keepdims=True)
    lse = jnp.log(jnp.sum(jnp.exp(logits - m), axis=-1, keepdims=True))
    act_ref[...] = logits - m - lse
    v = jnp.maximum(vpre_ref[...] + bv1_ref[...], 0.0)
    val_ref[...] = jnp.tanh(
        jnp.sum(v.reshape(64, 1) * wv2_ref[...], axis=0, keepdims=True)
        + bv2_ref[...])


def _head2(a_pre, ba1, Wa2, ba2, v_pre, bv1, Wv2, bv2):
    return pl.pallas_call(
        _head2_body,
        out_shape=[
            jax.ShapeDtypeStruct((1, N), jnp.float32),
            jax.ShapeDtypeStruct((1, 1), jnp.float32),
        ],
    )(a_pre, ba1.reshape(1, 64), Wa2, ba2.reshape(1, N),
      v_pre, bv1.reshape(1, 64), Wv2, bv2.reshape(1, 1))


def kernel(x, edge_index, W1, b1, W2, b2, Wa1, ba1, Wa2, ba2, Wv1, bv1, Wv2, bv2):
    h = jax.nn.relu(_gcn_xla(x, edge_index, W1, b1))
    h = jax.nn.relu(_gcn_xla(h, edge_index, W2, b2))
    flat = h.reshape(-1, FINAL)
    a_pre, v_pre = _head1(flat, Wa1, Wv1)
    x_act, x_val = _head2(a_pre, ba1, Wa2, ba2, v_pre, bv1, Wv2, bv2)
    return (x_act, x_val)
